# baseline (speedup 1.0000x reference)
import functools

import jax
import jax.numpy as jnp
from jax import lax
from jax.experimental import pallas as pl
from jax.experimental.pallas import tpu as pltpu
from jax.experimental.pallas import tpu_sc as plsc

D_MODEL = 2048
NC = 2
NS = 16
NW = NC * NS
B = 4 * 8192
B_PER_W = B // NW
CHUNK = 8
NCHUNK = B_PER_W // CHUNK

_mesh = plsc.VectorSubcoreMesh(
    core_axis_name="c", subcore_axis_name="s", num_cores=NC, num_subcores=NS
)


@functools.partial(
    pl.kernel,
    mesh=_mesh,
    out_type=jax.ShapeDtypeStruct((B, D_MODEL), jnp.float32),
    scratch_types=[
        pltpu.VMEM((B_PER_W,), jnp.int32),
        pltpu.VMEM((CHUNK, D_MODEL), jnp.float32),
        pltpu.VMEM((CHUNK, D_MODEL), jnp.float32),
        pltpu.SemaphoreType.DMA,
        pltpu.SemaphoreType.DMA,
        pltpu.SemaphoreType.DMA,
        pltpu.SemaphoreType.DMA,
    ],
)
def _gather_rows(
    table_hbm, idx_hbm, out_hbm, idx_v, rows0, rows1, g0, g1, w0, w1
):
    rows = (rows0, rows1)
    gsems = (g0, g1)
    wsems = (w0, w1)
    wid = lax.axis_index("s") * NC + lax.axis_index("c")
    base = wid * B_PER_W
    pltpu.sync_copy(idx_hbm.at[pl.ds(base, B_PER_W)], idx_v)

    def gather_start(c, b):
        pltpu.async_copy(
            table_hbm.at[idx_v.at[pl.ds(c * CHUNK, CHUNK)]], rows[b], gsems[b]
        )

    def gather_wait(b):
        pltpu.make_async_copy(
            table_hbm.at[pl.ds(0, CHUNK)], rows[b], gsems[b]
        ).wait()

    def write_start(c, b):
        pltpu.async_copy(
            rows[b], out_hbm.at[pl.ds(base + c * CHUNK, CHUNK)], wsems[b]
        )

    def write_wait(b):
        pltpu.make_async_copy(
            out_hbm.at[pl.ds(0, CHUNK)], rows[b], wsems[b]
        ).wait()

    gather_start(0, 0)
    gather_wait(0)
    write_start(0, 0)
    gather_start(1, 1)

    @pl.loop(1, NCHUNK - 1, step=2)
    def _(c):
        for b in range(2):
            i = c + b
            bb = (1 + b) % 2
            gather_wait(bb)
            write_start(i, bb)
            write_wait(b)
            gather_start(i + 1, b)

    gather_wait(1)
    write_start(NCHUNK - 1, 1)
    write_wait(0)
    write_wait(1)


def kernel(positions, table):
    idx = positions.reshape(-1).astype(jnp.int32)
    out = _gather_rows(table, idx)
    return out.reshape(*positions.shape, D_MODEL)


# speedup vs baseline: 1.2456x; 1.2456x over previous
import functools

import jax
import jax.numpy as jnp
from jax import lax
from jax.experimental import pallas as pl
from jax.experimental.pallas import tpu as pltpu
from jax.experimental.pallas import tpu_sc as plsc

D_MODEL = 2048
NC = 2
NS = 16
NW = NC * NS
B = 4 * 8192
B_PER_W = B // NW
CHUNK = 24
NFULL, TAIL = divmod(B_PER_W, CHUNK)
_CHUNKS = [(c * CHUNK, CHUNK) for c in range(NFULL)]
if TAIL:
    _CHUNKS.append((NFULL * CHUNK, TAIL))
NCH = len(_CHUNKS)
MAIN = ((NFULL - 2) // 2) * 2

_mesh = plsc.VectorSubcoreMesh(
    core_axis_name="c", subcore_axis_name="s", num_cores=NC, num_subcores=NS
)


@functools.partial(
    pl.kernel,
    mesh=_mesh,
    out_type=jax.ShapeDtypeStruct((B, D_MODEL), jnp.float32),
    scratch_types=[
        pltpu.VMEM((B_PER_W,), jnp.int32),
        pltpu.VMEM((CHUNK, D_MODEL), jnp.float32),
        pltpu.VMEM((CHUNK, D_MODEL), jnp.float32),
        pltpu.SemaphoreType.DMA,
        pltpu.SemaphoreType.DMA,
    ],
)
def _gather_rows(table_hbm, idx_hbm, out_hbm, idx_v, rows0, rows1, g0, g1):
    rows = (rows0, rows1)
    gsems = (g0, g1)
    wid = lax.axis_index("s") * NC + lax.axis_index("c")
    base = wid * B_PER_W
    pltpu.sync_copy(idx_hbm.at[pl.ds(base, B_PER_W)], idx_v)

    def gather_start(off, n, b):
        pltpu.async_copy(
            table_hbm.at[idx_v.at[pl.ds(off, n)]],
            rows[b].at[pl.ds(0, n)],
            gsems[b],
        )

    def gather_wait(n, b):
        pltpu.make_async_copy(
            table_hbm.at[pl.ds(0, n)], rows[b].at[pl.ds(0, n)], gsems[b]
        ).wait()

    def write_out(off, n, b):
        pltpu.sync_copy(
            rows[b].at[pl.ds(0, n)], out_hbm.at[pl.ds(base + off, n)]
        )

    gather_start(*_CHUNKS[0], 0)
    gather_start(*_CHUNKS[1], 1)

    @pl.loop(0, MAIN, step=2)
    def _(c):
        for b in range(2):
            gather_wait(CHUNK, b)
            write_out((c + b) * CHUNK, CHUNK, b)
            gather_start((c + b + 2) * CHUNK, CHUNK, b)

    for i in range(MAIN, NCH):
        b = i % 2
        off, n = _CHUNKS[i]
        gather_wait(n, b)
        write_out(off, n, b)
        if i + 2 < NCH:
            gather_start(*_CHUNKS[i + 2], b)


def kernel(positions, table):
    idx = positions.reshape(-1).astype(jnp.int32)
    out = _gather_rows(table, idx)
    return out.reshape(*positions.shape, D_MODEL)
